# Initial kernel scaffold; baseline (speedup 1.0000x reference)
import jax
import jax.numpy as jnp
from jax.experimental import pallas as pl


def kernel(src_val, src_state, dst_val, dst_state, W_route):
    raise NotImplementedError("write your pallas kernel here")


# speedup vs baseline: 16.9442x; 16.9442x over previous
import jax
import jax.numpy as jnp
from jax.experimental import pallas as pl
from jax.experimental.pallas import tpu as pltpu

_K = 64


def _fused_body(xv_ref, st_ref, dv_ref, ds_ref, w_ref, ov_ref, os_ref):
    s_idx = pl.program_id(1)
    x = xv_ref[0]
    w = w_ref[...]
    logits = jnp.dot(x, w, preferred_element_type=jnp.float32)

    u = jax.lax.bitcast_convert_type(logits, jnp.int32)
    key = jnp.where(u < 0, u ^ jnp.int32(0x7FFFFFFF), u)

    kk = jnp.int32(_K)
    cntpos = jnp.sum((key >= 0).astype(jnp.int32), axis=1, keepdims=True)
    base = jnp.where(cntpos >= kk, jnp.int32(0), jnp.int32(-(2 ** 31)))

    def body(i, b):
        cand = b | (jnp.int32(1) << (jnp.int32(30) - i))
        cnt = jnp.sum((key >= cand).astype(jnp.int32), axis=1, keepdims=True)
        return jnp.where(cnt >= kk, cand, b)

    base = jax.lax.fori_loop(0, 31, body, base)

    mask = key >= base
    rowmax = jnp.max(logits, axis=1, keepdims=True)
    e = jnp.where(mask, jnp.exp(logits - rowmax), 0.0)
    denom = jnp.sum(e, axis=1, keepdims=True)

    stt = st_ref[0]
    sp = jnp.maximum(stt, 0.0) + jnp.log(1.0 + jnp.exp(-jnp.abs(stt)))
    wts = e * (sp / denom)

    dv = jax.lax.dot_general(wts, x, (((0,), (0,)), ((), ())),
                             preferred_element_type=jnp.float32)
    dstt = jax.lax.dot_general(wts, stt, (((0,), (0,)), ((), ())),
                               preferred_element_type=jnp.float32)

    @pl.when(s_idx == 0)
    def _():
        ov_ref[0] = dv_ref[0] + dv
        os_ref[0] = ds_ref[0] + dstt

    @pl.when(s_idx != 0)
    def _():
        ov_ref[0] = ov_ref[0] + dv
        os_ref[0] = os_ref[0] + dstt


def kernel(src_val, src_state, dst_val, dst_state, W_route):
    B, S, D = src_val.shape
    N = W_route.shape[1]
    TS = min(256, S)
    grid = (B, S // TS)

    out_val, out_state = pl.pallas_call(
        _fused_body,
        grid=grid,
        in_specs=[
            pl.BlockSpec((1, TS, D), lambda b, s: (b, s, 0)),
            pl.BlockSpec((1, TS, 1), lambda b, s: (b, s, 0)),
            pl.BlockSpec((1, N, D), lambda b, s: (b, 0, 0)),
            pl.BlockSpec((1, N, 1), lambda b, s: (b, 0, 0)),
            pl.BlockSpec((D, N), lambda b, s: (0, 0)),
        ],
        out_specs=[
            pl.BlockSpec((1, N, D), lambda b, s: (b, 0, 0)),
            pl.BlockSpec((1, N, 1), lambda b, s: (b, 0, 0)),
        ],
        out_shape=[
            jax.ShapeDtypeStruct((B, N, D), jnp.float32),
            jax.ShapeDtypeStruct((B, N, 1), jnp.float32),
        ],
        compiler_params=pltpu.CompilerParams(
            dimension_semantics=("arbitrary", "arbitrary"),
        ),
    )(src_val, src_state[..., None], dst_val, dst_state[..., None], W_route)
    return out_val, out_state[..., 0]
